# baseline (speedup 1.0000x reference)
import functools

import jax
import jax.numpy as jnp
from jax import lax
from jax.experimental import pallas as pl
from jax.experimental.pallas import tpu as pltpu
from jax.experimental.pallas import tpu_sc as plsc

NL = 10000
NLP = 10240
NQ = 50000
NQP = 50176
D = 256

B = 512
NW = 32
QPW = NQP // NW
C = 56
NCHUNK = QPW // C


def _top3_body(ph_ref, plt_ref, idx_ref, w_ref):
    ph = ph_ref[...]
    phx, phy, phz = ph[:, 0:1], ph[:, 1:2], ph[:, 2:3]
    plx = plt_ref[0:1, :]
    ply = plt_ref[1:2, :]
    plz = plt_ref[2:3, :]
    dx = phx - plx
    dy = phy - ply
    dz = phz - plz
    d2 = dx * dx + dy * dy + dz * dz
    lane = lax.broadcasted_iota(jnp.int32, (B, NLP), 1).astype(jnp.float32)
    idxs, vals = [], []
    cur = d2
    for k in range(3):
        m = jnp.min(cur, axis=1, keepdims=True)
        eq = cur == m
        ik = jnp.min(jnp.where(eq, lane, float(NLP)), axis=1, keepdims=True)
        idxs.append(ik)
        vals.append(m)
        if k < 2:
            cur = jnp.where(eq, jnp.inf, cur)
    val = jnp.concatenate(vals, axis=1)
    w = 1.0 / jnp.maximum(val, 1e-16)
    den = jnp.sum(w, axis=1, keepdims=True)
    wn = w / den
    idx_ref[...] = jnp.concatenate(idxs, axis=1).astype(jnp.int32)
    w_ref[...] = jnp.concatenate(
        [jnp.broadcast_to(wn[:, k:k + 1], (wn.shape[0], 16)) for k in range(3)],
        axis=1)


def _top3(ph, plt):
    n = ph.shape[0]
    return pl.pallas_call(
        _top3_body,
        grid=(n // B,),
        in_specs=[
            pl.BlockSpec((B, 3), lambda i: (i, 0)),
            pl.BlockSpec((8, NLP), lambda i: (0, 0)),
        ],
        out_specs=[
            pl.BlockSpec((B, 3), lambda i: (i, 0)),
            pl.BlockSpec((B, 48), lambda i: (i, 0)),
        ],
        out_shape=[
            jax.ShapeDtypeStruct((n, 3), jnp.int32),
            jax.ShapeDtypeStruct((n, 48), jnp.float32),
        ],
    )(ph, plt)


def _sc_combine_body(qpw, x_hbm, idx_hbm, w_hbm, out_hbm,
                     idx0, idx1, w0, w1, rows0, rows1, out0, out1, sem0, sem1):
    wid = lax.axis_index("s") * 2 + lax.axis_index("c")
    qbase = wid * qpw
    nchunk = qpw // C

    def fetch(ci, idx_v, w_v, rows_v, sem):
        q0 = qbase + ci * C
        e0 = pl.multiple_of(3 * q0, 8)
        pltpu.sync_copy(idx_hbm.at[pl.ds(e0, 3 * C)], idx_v)
        pltpu.sync_copy(w_hbm.at[pl.ds(q0, C)], w_v)
        return pltpu.async_copy(x_hbm.at[idx_v], rows_v, sem)

    def compute(ci, w_v, rows_v, out_v):
        @plsc.parallel_loop(0, C, 1, unroll=4)
        def qstep(q):
            b = 3 * q
            wa = w_v[q, pl.ds(0, 16)]
            wb = w_v[q, pl.ds(16, 16)]
            wc = w_v[q, pl.ds(32, 16)]
            for f in range(D // 16):
                s = pl.ds(16 * f, 16)
                out_v[q, s] = (wa * rows_v[b, s] + wb * rows_v[b + 1, s]
                               + wc * rows_v[b + 2, s])

        q0 = qbase + ci * C
        pltpu.sync_copy(out_v, out_hbm.at[pl.ds(q0, C)])

    def pair(g, carry):
        c0 = 2 * g
        c1 = 2 * g + 1
        h0 = fetch(c0, idx0, w0, rows0, sem0)
        h1 = fetch(c1, idx1, w1, rows1, sem1)
        h0.wait()
        compute(c0, w0, rows0, out0)
        h1.wait()
        compute(c1, w1, rows1, out1)
        return carry

    lax.fori_loop(0, nchunk // 2, pair, 0)


@functools.cache
def _sc_combine(n):
    mesh = plsc.VectorSubcoreMesh(
        core_axis_name="c", subcore_axis_name="s", num_cores=2, num_subcores=16)
    return pl.kernel(
        functools.partial(_sc_combine_body, n // NW),
        out_type=jax.ShapeDtypeStruct((n, D), jnp.float32),
        mesh=mesh,
        scratch_types=[
            pltpu.VMEM((3 * C,), jnp.int32),
            pltpu.VMEM((3 * C,), jnp.int32),
            pltpu.VMEM((C, 48), jnp.float32),
            pltpu.VMEM((C, 48), jnp.float32),
            pltpu.VMEM((3 * C, D), jnp.float32),
            pltpu.VMEM((3 * C, D), jnp.float32),
            pltpu.VMEM((C, D), jnp.float32),
            pltpu.VMEM((C, D), jnp.float32),
            pltpu.SemaphoreType.DMA,
            pltpu.SemaphoreType.DMA,
        ],
    )


def kernel(x, pos_l, pos_h):
    ph = jnp.pad(pos_h, ((0, NQP - NQ), (0, 0)))
    plt = jnp.pad(pos_l.T, ((0, 5), (0, NLP - NL)),
                  constant_values=jnp.float32(jnp.inf))
    na = NQP // 2
    idx_a, w_a = _top3(ph[:na], plt)
    out_a = _sc_combine(na)(x, idx_a.reshape(-1), w_a)
    idx_b, w_b = _top3(ph[na:], plt)
    out_b = _sc_combine(NQP - na)(x, idx_b.reshape(-1), w_b)
    return jnp.concatenate([out_a, out_b[:NQ - na]], axis=0)


# speedup vs baseline: 1.0031x; 1.0031x over previous
import functools

import jax
import jax.numpy as jnp
from jax import lax
from jax.experimental import pallas as pl
from jax.experimental.pallas import tpu as pltpu
from jax.experimental.pallas import tpu_sc as plsc

NL = 10000
NLP = 10240
NQ = 50000
NQP = 50176
D = 256

B = 512
NW = 32
QPW = NQP // NW
C = 56
NCHUNK = QPW // C


def _top3_body(ph_ref, plt_ref, idx_ref, w_ref):
    ph = ph_ref[...]
    phx, phy, phz = ph[:, 0:1], ph[:, 1:2], ph[:, 2:3]
    plx = plt_ref[0:1, :]
    ply = plt_ref[1:2, :]
    plz = plt_ref[2:3, :]
    dx = phx - plx
    dy = phy - ply
    dz = phz - plz
    d2 = dx * dx + dy * dy + dz * dz
    lane = lax.broadcasted_iota(jnp.int32, (B, NLP), 1).astype(jnp.float32)
    idxs, vals = [], []
    cur = d2
    for k in range(3):
        m = jnp.min(cur, axis=1, keepdims=True)
        eq = cur == m
        ik = jnp.min(jnp.where(eq, lane, float(NLP)), axis=1, keepdims=True)
        idxs.append(ik)
        vals.append(m)
        if k < 2:
            cur = jnp.where(eq, jnp.inf, cur)
    val = jnp.concatenate(vals, axis=1)
    w = 1.0 / jnp.maximum(val, 1e-16)
    den = jnp.sum(w, axis=1, keepdims=True)
    wn = w / den
    idx_ref[...] = jnp.concatenate(idxs, axis=1).astype(jnp.int32)
    w_ref[...] = jnp.concatenate(
        [jnp.broadcast_to(wn[:, k:k + 1], (wn.shape[0], 16)) for k in range(3)],
        axis=1)


def _top3(ph, plt):
    n = ph.shape[0]
    return pl.pallas_call(
        _top3_body,
        grid=(n // B,),
        in_specs=[
            pl.BlockSpec((B, 3), lambda i: (i, 0)),
            pl.BlockSpec((8, NLP), lambda i: (0, 0)),
        ],
        out_specs=[
            pl.BlockSpec((B, 3), lambda i: (i, 0)),
            pl.BlockSpec((B, 48), lambda i: (i, 0)),
        ],
        out_shape=[
            jax.ShapeDtypeStruct((n, 3), jnp.int32),
            jax.ShapeDtypeStruct((n, 48), jnp.float32),
        ],
    )(ph, plt)


def _sc_combine_body(qpw, x_hbm, idx_hbm, w_hbm, out_hbm,
                     idx0, idx1, w0, w1, rows0, rows1, out0, out1, sem0, sem1):
    wid = lax.axis_index("s") * 2 + lax.axis_index("c")
    qbase = wid * qpw
    nchunk = qpw // C

    def fetch(ci, idx_v, w_v, rows_v, sem):
        q0 = qbase + ci * C
        e0 = pl.multiple_of(3 * q0, 8)
        pltpu.sync_copy(idx_hbm.at[pl.ds(e0, 3 * C)], idx_v)
        pltpu.sync_copy(w_hbm.at[pl.ds(q0, C)], w_v)
        return pltpu.async_copy(x_hbm.at[idx_v], rows_v, sem)

    def compute(ci, w_v, rows_v, out_v):
        @plsc.parallel_loop(0, C, 1, unroll=2)
        def qstep(q):
            b = 3 * q
            wa = w_v[q, pl.ds(0, 16)]
            wb = w_v[q, pl.ds(16, 16)]
            wc = w_v[q, pl.ds(32, 16)]
            for f in range(D // 16):
                s = pl.ds(16 * f, 16)
                out_v[q, s] = (wa * rows_v[b, s] + wb * rows_v[b + 1, s]
                               + wc * rows_v[b + 2, s])

        q0 = qbase + ci * C
        pltpu.sync_copy(out_v, out_hbm.at[pl.ds(q0, C)])

    def pair(g, carry):
        c0 = 2 * g
        c1 = 2 * g + 1
        h0 = fetch(c0, idx0, w0, rows0, sem0)
        h1 = fetch(c1, idx1, w1, rows1, sem1)
        h0.wait()
        compute(c0, w0, rows0, out0)
        h1.wait()
        compute(c1, w1, rows1, out1)
        return carry

    lax.fori_loop(0, nchunk // 2, pair, 0)


@functools.cache
def _sc_combine(n):
    mesh = plsc.VectorSubcoreMesh(
        core_axis_name="c", subcore_axis_name="s", num_cores=2, num_subcores=16)
    return pl.kernel(
        functools.partial(_sc_combine_body, n // NW),
        out_type=jax.ShapeDtypeStruct((n, D), jnp.float32),
        mesh=mesh,
        scratch_types=[
            pltpu.VMEM((3 * C,), jnp.int32),
            pltpu.VMEM((3 * C,), jnp.int32),
            pltpu.VMEM((C, 48), jnp.float32),
            pltpu.VMEM((C, 48), jnp.float32),
            pltpu.VMEM((3 * C, D), jnp.float32),
            pltpu.VMEM((3 * C, D), jnp.float32),
            pltpu.VMEM((C, D), jnp.float32),
            pltpu.VMEM((C, D), jnp.float32),
            pltpu.SemaphoreType.DMA,
            pltpu.SemaphoreType.DMA,
        ],
    )


def kernel(x, pos_l, pos_h):
    ph = jnp.pad(pos_h, ((0, NQP - NQ), (0, 0)))
    plt = jnp.pad(pos_l.T, ((0, 5), (0, NLP - NL)),
                  constant_values=jnp.float32(jnp.inf))
    na = NQP // 2
    idx_a, w_a = _top3(ph[:na], plt)
    out_a = _sc_combine(na)(x, idx_a.reshape(-1), w_a)
    idx_b, w_b = _top3(ph[na:], plt)
    out_b = _sc_combine(NQP - na)(x, idx_b.reshape(-1), w_b)
    return jnp.concatenate([out_a, out_b[:NQ - na]], axis=0)


# speedup vs baseline: 1.0090x; 1.0059x over previous
import functools

import jax
import jax.numpy as jnp
from jax import lax
from jax.experimental import pallas as pl
from jax.experimental.pallas import tpu as pltpu
from jax.experimental.pallas import tpu_sc as plsc

NL = 10000
NLP = 10240
NQ = 50000
NQP = 50176
D = 256

B = 512
NW = 32
QPW = NQP // NW
C = 56
NCHUNK = QPW // C


def _top3_body(ph_ref, plt_ref, idx_ref, w_ref):
    ph = ph_ref[...]
    phx, phy, phz = ph[:, 0:1], ph[:, 1:2], ph[:, 2:3]
    plx = plt_ref[0:1, :]
    ply = plt_ref[1:2, :]
    plz = plt_ref[2:3, :]
    dx = phx - plx
    dy = phy - ply
    dz = phz - plz
    d2 = dx * dx + dy * dy + dz * dz
    lane = lax.broadcasted_iota(jnp.int32, (B, NLP), 1).astype(jnp.float32)
    idxs, vals = [], []
    cur = d2
    for k in range(3):
        m = jnp.min(cur, axis=1, keepdims=True)
        eq = cur == m
        ik = jnp.min(jnp.where(eq, lane, float(NLP)), axis=1, keepdims=True)
        idxs.append(ik)
        vals.append(m)
        if k < 2:
            cur = jnp.where(eq, jnp.inf, cur)
    val = jnp.concatenate(vals, axis=1)
    w = 1.0 / jnp.maximum(val, 1e-16)
    den = jnp.sum(w, axis=1, keepdims=True)
    wn = w / den
    idx_ref[...] = jnp.concatenate(idxs, axis=1).astype(jnp.int32)
    w_ref[...] = jnp.concatenate(
        [jnp.broadcast_to(wn[:, k:k + 1], (wn.shape[0], 16)) for k in range(3)],
        axis=1)


def _top3(ph, plt):
    n = ph.shape[0]
    return pl.pallas_call(
        _top3_body,
        grid=(n // B,),
        in_specs=[
            pl.BlockSpec((B, 3), lambda i: (i, 0)),
            pl.BlockSpec((8, NLP), lambda i: (0, 0)),
        ],
        out_specs=[
            pl.BlockSpec((B, 3), lambda i: (i, 0)),
            pl.BlockSpec((B, 48), lambda i: (i, 0)),
        ],
        out_shape=[
            jax.ShapeDtypeStruct((n, 3), jnp.int32),
            jax.ShapeDtypeStruct((n, 48), jnp.float32),
        ],
    )(ph, plt)


def _sc_combine_body(qpw, x_hbm, idx_hbm, w_hbm, out_hbm,
                     idx0, idx1, w0, w1, rows0, rows1, out0, out1, sem0, sem1):
    wid = lax.axis_index("s") * 2 + lax.axis_index("c")
    qbase = wid * qpw
    nchunk = qpw // C

    def fetch(ci, idx_v, w_v, rows_v, sem):
        q0 = qbase + ci * C
        e0 = pl.multiple_of(3 * q0, 8)
        pltpu.sync_copy(idx_hbm.at[pl.ds(e0, 3 * C)], idx_v)
        pltpu.sync_copy(w_hbm.at[pl.ds(q0, C)], w_v)
        return pltpu.async_copy(x_hbm.at[idx_v], rows_v, sem)

    def compute(ci, w_v, rows_v, out_v):
        @plsc.parallel_loop(0, C, 1, unroll=2)
        def qstep(q):
            b = 3 * q
            wa = w_v[q, pl.ds(0, 16)]
            wb = w_v[q, pl.ds(16, 16)]
            wc = w_v[q, pl.ds(32, 16)]
            for f in range(D // 16):
                s = pl.ds(16 * f, 16)
                out_v[q, s] = (wa * rows_v[b, s] + wb * rows_v[b + 1, s]
                               + wc * rows_v[b + 2, s])

        q0 = qbase + ci * C
        pltpu.sync_copy(out_v, out_hbm.at[pl.ds(q0, C)])

    def pair(g, carry):
        c0 = 2 * g
        c1 = 2 * g + 1
        h0 = fetch(c0, idx0, w0, rows0, sem0)
        h1 = fetch(c1, idx1, w1, rows1, sem1)
        h0.wait()
        compute(c0, w0, rows0, out0)
        h1.wait()
        compute(c1, w1, rows1, out1)
        return carry

    lax.fori_loop(0, nchunk // 2, pair, 0)


@functools.cache
def _sc_combine(n):
    mesh = plsc.VectorSubcoreMesh(
        core_axis_name="c", subcore_axis_name="s", num_cores=2, num_subcores=16)
    return pl.kernel(
        functools.partial(_sc_combine_body, n // NW),
        out_type=jax.ShapeDtypeStruct((n, D), jnp.float32),
        mesh=mesh,
        scratch_types=[
            pltpu.VMEM((3 * C,), jnp.int32),
            pltpu.VMEM((3 * C,), jnp.int32),
            pltpu.VMEM((C, 48), jnp.float32),
            pltpu.VMEM((C, 48), jnp.float32),
            pltpu.VMEM((3 * C, D), jnp.float32),
            pltpu.VMEM((3 * C, D), jnp.float32),
            pltpu.VMEM((C, D), jnp.float32),
            pltpu.VMEM((C, D), jnp.float32),
            pltpu.SemaphoreType.DMA,
            pltpu.SemaphoreType.DMA,
        ],
    )


def kernel(x, pos_l, pos_h):
    ph = jnp.pad(pos_h, ((0, NQP - NQ), (0, 0)))
    plt = jnp.pad(pos_l.T, ((0, 5), (0, NLP - NL)),
                  constant_values=jnp.float32(jnp.inf))
    na = NQP // 2
    idx_a, w_a = _top3(ph[:na], plt)
    out_a = _sc_combine(na)(x, idx_a.reshape(-1), w_a)
    idx_b, w_b = _top3(ph[na:], plt)
    out_b = _sc_combine(NQP - na)(x, idx_b.reshape(-1), w_b)
    return jnp.concatenate([out_a, out_b], axis=0)[:NQ]


# speedup vs baseline: 1.1470x; 1.1367x over previous
import functools

import jax
import jax.numpy as jnp
from jax import lax
from jax.experimental import pallas as pl
from jax.experimental.pallas import tpu as pltpu
from jax.experimental.pallas import tpu_sc as plsc

NL = 10000
NLP = 10240
NQ = 50000
NQP = 50176
D = 256

B = 512
NW = 32
QPW = NQP // NW
C = 56
NCHUNK = QPW // C


def _top3_body(ph_ref, plt_ref, idx_ref, w_ref):
    ph = ph_ref[...]
    phx, phy, phz = ph[:, 0:1], ph[:, 1:2], ph[:, 2:3]
    plx = plt_ref[0:1, :]
    ply = plt_ref[1:2, :]
    plz = plt_ref[2:3, :]
    dx = phx - plx
    dy = phy - ply
    dz = phz - plz
    d2 = dx * dx + dy * dy + dz * dz
    lane = lax.broadcasted_iota(jnp.int32, (B, NLP), 1).astype(jnp.float32)
    idxs, vals = [], []
    cur = d2
    for k in range(3):
        m = jnp.min(cur, axis=1, keepdims=True)
        eq = cur == m
        ik = jnp.min(jnp.where(eq, lane, float(NLP)), axis=1, keepdims=True)
        idxs.append(ik)
        vals.append(m)
        if k < 2:
            cur = jnp.where(eq, jnp.inf, cur)
    val = jnp.concatenate(vals, axis=1)
    w = 1.0 / jnp.maximum(val, 1e-16)
    den = jnp.sum(w, axis=1, keepdims=True)
    wn = w / den
    idx_ref[...] = jnp.concatenate(idxs, axis=1).astype(jnp.int32)
    w_ref[...] = jnp.concatenate(
        [jnp.broadcast_to(wn[:, k:k + 1], (wn.shape[0], 16)) for k in range(3)],
        axis=1)


def _top3(ph, plt):
    n = ph.shape[0]
    return pl.pallas_call(
        _top3_body,
        grid=(n // B,),
        in_specs=[
            pl.BlockSpec((B, 3), lambda i: (i, 0)),
            pl.BlockSpec((8, NLP), lambda i: (0, 0)),
        ],
        out_specs=[
            pl.BlockSpec((B, 3), lambda i: (i, 0)),
            pl.BlockSpec((B, 48), lambda i: (i, 0)),
        ],
        out_shape=[
            jax.ShapeDtypeStruct((n, 3), jnp.int32),
            jax.ShapeDtypeStruct((n, 48), jnp.float32),
        ],
    )(ph, plt)


def _sc_combine_body(qpw, x_hbm, idx_hbm, w_hbm, out_hbm,
                     idx0, idx1, w0, w1, rows0, rows1, out0, out1, sem0, sem1):
    wid = lax.axis_index("s") * 2 + lax.axis_index("c")
    qbase = wid * qpw
    nchunk = qpw // C

    def fetch(ci, idx_v, w_v, rows_v, sem):
        q0 = qbase + ci * C
        e0 = pl.multiple_of(3 * q0, 8)
        pltpu.sync_copy(idx_hbm.at[pl.ds(e0, 3 * C)], idx_v)
        pltpu.sync_copy(w_hbm.at[pl.ds(q0, C)], w_v)
        return pltpu.async_copy(x_hbm.at[idx_v], rows_v, sem)

    def compute(ci, w_v, rows_v, out_v):
        @plsc.parallel_loop(0, C, 1, unroll=2)
        def qstep(q):
            b = 3 * q
            wa = w_v[q, pl.ds(0, 16)]
            wb = w_v[q, pl.ds(16, 16)]
            wc = w_v[q, pl.ds(32, 16)]
            for f in range(D // 16):
                s = pl.ds(16 * f, 16)
                out_v[q, s] = (wa * rows_v[b, s] + wb * rows_v[b + 1, s]
                               + wc * rows_v[b + 2, s])

        q0 = qbase + ci * C
        pltpu.sync_copy(out_v, out_hbm.at[pl.ds(q0, C)])

    def pair(g, carry):
        c0 = 2 * g
        c1 = 2 * g + 1
        h0 = fetch(c0, idx0, w0, rows0, sem0)
        h1 = fetch(c1, idx1, w1, rows1, sem1)
        h0.wait()
        compute(c0, w0, rows0, out0)
        h1.wait()
        compute(c1, w1, rows1, out1)
        return carry

    lax.fori_loop(0, nchunk // 2, pair, 0)


def _sc_knn_body(qsc, phb_hbm, plt_hbm, idx_hbm, w_hbm,
                 plv, phbuf, idxbuf, wbuf):
    wid = lax.axis_index("s") * 2 + lax.axis_index("c")
    qb = wid * qsc
    pltpu.sync_copy(plt_hbm.at[pl.ds(0, 3)], plv)
    pltpu.sync_copy(phb_hbm.at[pl.ds(qb, qsc)], phbuf)
    lanes = lax.iota(jnp.int32, 16)
    inf16 = jnp.full((16,), jnp.inf, jnp.float32)
    zero16 = jnp.zeros((16,), jnp.int32)

    def qstep(q, carry):
        phx = phbuf[q, pl.ds(0, 16)]
        phy = phbuf[q, pl.ds(16, 16)]
        phz = phbuf[q, pl.ds(32, 16)]

        def jstep(j, c):
            m1, m2, m3, i1, i2, i3 = c
            s = pl.ds(16 * j, 16)
            dx = phx - plv[0, s]
            dy = phy - plv[1, s]
            dz = phz - plv[2, s]
            v = dx * dx + dy * dy + dz * dz
            iv = 16 * j + lanes
            c1 = v < m1
            c2 = v < m2
            c3 = v < m3
            nm3 = jnp.where(c3, jnp.where(c2, m2, v), m3)
            ni3 = jnp.where(c3, jnp.where(c2, i2, iv), i3)
            nm2 = jnp.where(c2, jnp.where(c1, m1, v), m2)
            ni2 = jnp.where(c2, jnp.where(c1, i1, iv), i2)
            nm1 = jnp.where(c1, v, m1)
            ni1 = jnp.where(c1, iv, i1)
            return (nm1, nm2, nm3, ni1, ni2, ni3)

        mv = [inf16, inf16, inf16]
        iv_ = [zero16, zero16, zero16]
        mv[0], mv[1], mv[2], iv_[0], iv_[1], iv_[2] = lax.fori_loop(
            0, NLP // 16, jstep, (mv[0], mv[1], mv[2], iv_[0], iv_[1], iv_[2]))

        gdn = lax.GatherDimensionNumbers(
            offset_dims=(), collapsed_slice_dims=(0,), start_index_map=(0,))

        def allmin(v):
            for sh in (8, 4, 2, 1):
                p = lax.gather(v, (lanes ^ sh)[:, None], gdn, (1,),
                               mode=lax.GatherScatterMode.PROMISE_IN_BOUNDS)
                v = jnp.minimum(v, p)
            return v

        gvals, gidx = [], []
        for k in range(3):
            gk = allmin(jnp.minimum(jnp.minimum(mv[0], mv[1]), mv[2]))
            ik = allmin(jnp.minimum(
                jnp.minimum(jnp.where(mv[0] == gk, iv_[0], NLP),
                            jnp.where(mv[1] == gk, iv_[1], NLP)),
                jnp.where(mv[2] == gk, iv_[2], NLP)))
            gvals.append(gk)
            gidx.append(ik)
            if k < 2:
                for t in range(3):
                    mv[t] = jnp.where((mv[t] == gk) & (iv_[t] == ik),
                                      jnp.inf, mv[t])
        w = [1.0 / jnp.maximum(gvals[t], 1e-16) for t in range(3)]
        den = w[0] + w[1] + w[2]
        for t in range(3):
            wbuf[q, pl.ds(16 * t, 16)] = w[t] / den
        ivec = jnp.where(lanes == 0, gidx[0],
                         jnp.where(lanes == 1, gidx[1], gidx[2]))
        idxbuf[pl.ds(3 * q, 16)] = ivec
        return carry

    lax.fori_loop(0, qsc, qstep, 0)
    pltpu.sync_copy(idxbuf.at[pl.ds(0, 3 * qsc)],
                    idx_hbm.at[pl.ds(pl.multiple_of(3 * qb, 8), 3 * qsc)])
    pltpu.sync_copy(wbuf, w_hbm.at[pl.ds(qb, qsc)])


@functools.cache
def _sc_knn(n):
    mesh = plsc.VectorSubcoreMesh(
        core_axis_name="c", subcore_axis_name="s", num_cores=2, num_subcores=16)
    qsc = n // NW
    return pl.kernel(
        functools.partial(_sc_knn_body, qsc),
        out_type=[
            jax.ShapeDtypeStruct((3 * n,), jnp.int32),
            jax.ShapeDtypeStruct((n, 48), jnp.float32),
        ],
        mesh=mesh,
        scratch_types=[
            pltpu.VMEM((3, NLP), jnp.float32),
            pltpu.VMEM((qsc, 48), jnp.float32),
            pltpu.VMEM((3 * qsc + 16,), jnp.int32),
            pltpu.VMEM((qsc, 48), jnp.float32),
        ],
    )


@functools.cache
def _sc_combine(n):
    mesh = plsc.VectorSubcoreMesh(
        core_axis_name="c", subcore_axis_name="s", num_cores=2, num_subcores=16)
    return pl.kernel(
        functools.partial(_sc_combine_body, n // NW),
        out_type=jax.ShapeDtypeStruct((n, D), jnp.float32),
        mesh=mesh,
        scratch_types=[
            pltpu.VMEM((3 * C,), jnp.int32),
            pltpu.VMEM((3 * C,), jnp.int32),
            pltpu.VMEM((C, 48), jnp.float32),
            pltpu.VMEM((C, 48), jnp.float32),
            pltpu.VMEM((3 * C, D), jnp.float32),
            pltpu.VMEM((3 * C, D), jnp.float32),
            pltpu.VMEM((C, D), jnp.float32),
            pltpu.VMEM((C, D), jnp.float32),
            pltpu.SemaphoreType.DMA,
            pltpu.SemaphoreType.DMA,
        ],
    )


def kernel(x, pos_l, pos_h):
    ph = jnp.pad(pos_h, ((0, NQP - NQ), (0, 0)))
    plt = jnp.pad(pos_l.T, ((0, 5), (0, NLP - NL)),
                  constant_values=jnp.float32(jnp.inf))
    nsc = 7168
    ntc = NQP - nsc
    half = ntc // 2
    ph_t = ph[ntc:]
    phb = jnp.concatenate(
        [jnp.broadcast_to(ph_t[:, k:k + 1], (nsc, 16)) for k in range(3)],
        axis=1)
    idx_c, w_c = _sc_knn(nsc)(phb, plt)
    idx_a, w_a = _top3(ph[:half], plt)
    out_a = _sc_combine(half)(x, idx_a.reshape(-1), w_a)
    idx_b, w_b = _top3(ph[half:ntc], plt)
    out_b = _sc_combine(half)(x, idx_b.reshape(-1), w_b)
    out_c = _sc_combine(nsc)(x, idx_c, w_c)
    return jnp.concatenate([out_a, out_b, out_c], axis=0)[:NQ]
